# Initial kernel scaffold; baseline (speedup 1.0000x reference)
import jax
import jax.numpy as jnp
from jax.experimental import pallas as pl


def kernel(x, edge_index, W, b):
    raise NotImplementedError("write your pallas kernel here")


# speedup vs baseline: 20.7087x; 20.7087x over previous
import functools

import jax
import jax.numpy as jnp
from jax import lax
from jax.experimental import pallas as pl
from jax.experimental.pallas import tpu as pltpu
from jax.experimental.pallas import tpu_sc as plsc

N = 10000
D = 128
HALF = 64
K = 50
ALPHA = 0.1
NS = 16
NC = 2
CH = 128
NPT = N // NS
NROW = 125
NCHUNK = NPT // NROW
N_AGG = 10240
DEGW = 16


def _rsqrt_newton(d):
    i = plsc.bitcast(d, jnp.int32)
    i = jnp.int32(0x5F3759DF) - (i >> 1)
    y = plsc.bitcast(i, jnp.float32)
    for _ in range(3):
        y = y * (1.5 - 0.5 * d * y * y)
    return y


def _prop_body(h_hbm, src_hbm, dst_hbm, z_hbm, zt_hbm,
               srcv, dstv, ebA, ebB, ones16, degb, dinvv,
               hb, agg_sh, deg_sh, semA, semB):
    c = lax.axis_index("c")
    s = lax.axis_index("s")
    cN = c * N
    cpt = srcv.shape[0]

    pltpu.sync_copy(src_hbm.at[s], srcv)
    pltpu.sync_copy(dst_hbm.at[s], dstv)

    @pl.loop(0, cpt)
    def _shift(j):
        for q in range(CH // 16):
            sl = pl.ds(q * 16, 16)
            srcv[j, sl] = srcv[j, sl] + cN

    zero = jnp.zeros((16,), jnp.float32)
    e0 = jnp.where(lax.iota(jnp.int32, 16) == 0, 1.0, 0.0)

    @pl.loop(0, CH)
    def _fill(i):
        for q in range(HALF // 16):
            ebA[i, pl.ds(q * 16, 16)] = zero
        ones16[i, :] = e0

    @pl.loop(0, 160)
    def _zdeg(i):
        degb[i, :] = zero

    @pl.loop(0, 4)
    def _zdeg2(k):
        pltpu.sync_copy(degb, deg_sh.at[pl.ds(s * 640 + k * 160, 160)])

    @pl.loop(0, 5)
    def _zagg(k):
        pltpu.sync_copy(ebA, agg_sh.at[pl.ds(s * 640 + k * 128, 128)])

    plsc.subcore_barrier()

    @pl.loop(0, cpt)
    def _deg(j):
        pltpu.sync_copy(ones16, deg_sh.at[dstv.at[j]], add=True)

    plsc.subcore_barrier()

    lane = lax.iota(jnp.int32, 16)
    zlane = jnp.zeros((16,), jnp.int32)

    @pl.loop(0, 4)
    def _dinv(cc):
        pltpu.sync_copy(deg_sh.at[pl.ds(s * NPT + cc * 160, 160)], degb)

        @pl.loop(0, 10)
        def _grp(g):
            dvals = plsc.load_gather(degb, [g * 16 + lane, zlane]) + 1.0
            dinvv[pl.ds(cc * 160 + g * 16, 16)] = _rsqrt_newton(dvals)

    @pl.loop(0, NCHUNK)
    def _init(k):
        r0 = s * NPT + k * NROW
        pltpu.sync_copy(h_hbm.at[pl.ds(cN + r0, NROW)], hb)

        @pl.loop(0, NROW)
        def _rows(i):
            dv = dinvv[pl.ds(k * NROW + i, 16)][0]
            for q in range(HALF // 16):
                sl = pl.ds(q * 16, 16)
                hb[i, sl] = dv * hb[i, sl]

        pltpu.sync_copy(hb, zt_hbm.at[pl.ds(cN + r0, NROW)])

    plsc.subcore_barrier()

    @pl.loop(0, K)
    def _step(t):
        pltpu.async_copy(zt_hbm.at[srcv.at[0]], ebA, semA)

        @pl.loop(0, cpt // 2)
        def _edges(kk):
            j = 2 * kk
            pltpu.async_copy(zt_hbm.at[srcv.at[j + 1]], ebB, semB)
            pltpu.make_async_copy(zt_hbm.at[srcv.at[j]], ebA, semA).wait()
            pltpu.sync_copy(ebA, agg_sh.at[dstv.at[j]], add=True)

            @pl.when(j + 2 < cpt)
            def _pref():
                pltpu.async_copy(zt_hbm.at[srcv.at[j + 2]], ebA, semA)

            pltpu.make_async_copy(zt_hbm.at[srcv.at[j + 1]], ebB, semB).wait()
            pltpu.sync_copy(ebB, agg_sh.at[dstv.at[j + 1]], add=True)

        plsc.subcore_barrier()

        @pl.loop(0, NCHUNK)
        def _nodes(k):
            r0 = s * NPT + k * NROW
            pltpu.sync_copy(agg_sh.at[pl.ds(r0, NROW)], ebA.at[pl.ds(0, NROW)])
            pltpu.sync_copy(zt_hbm.at[pl.ds(cN + r0, NROW)], ebB.at[pl.ds(0, NROW)])
            pltpu.sync_copy(h_hbm.at[pl.ds(cN + r0, NROW)], hb)

            @pl.loop(0, NROW)
            def _rows(i):
                dv = dinvv[pl.ds(k * NROW + i, 16)][0]
                a9 = (1.0 - ALPHA) * dv
                for q in range(HALF // 16):
                    sl = pl.ds(q * 16, 16)
                    zv = a9 * (ebA[i, sl] + ebB[i, sl]) + ALPHA * hb[i, sl]
                    hb[i, sl] = zv
                    ebB[i, sl] = dv * zv
                    ebA[i, sl] = zero

            pltpu.sync_copy(ebB.at[pl.ds(0, NROW)], zt_hbm.at[pl.ds(cN + r0, NROW)])
            pltpu.sync_copy(hb, z_hbm.at[pl.ds(cN + r0, NROW)])
            pltpu.sync_copy(ebA.at[pl.ds(0, NROW)], agg_sh.at[pl.ds(r0, NROW)])

        plsc.subcore_barrier()


def _propagate(h2, srcp, dstp):
    cpt = srcp.shape[1]
    mesh = plsc.VectorSubcoreMesh(core_axis_name="c", subcore_axis_name="s")
    fn = pl.kernel(
        _prop_body,
        out_type=(jax.ShapeDtypeStruct((NC * N, HALF), jnp.float32),
                  jax.ShapeDtypeStruct((NC * N, HALF), jnp.float32)),
        mesh=mesh,
        scratch_types=[
            pltpu.VMEM((cpt, CH), jnp.int32),
            pltpu.VMEM((cpt, CH), jnp.int32),
            pltpu.VMEM((CH, HALF), jnp.float32),
            pltpu.VMEM((CH, HALF), jnp.float32),
            pltpu.VMEM((CH, DEGW), jnp.float32),
            pltpu.VMEM((160, DEGW), jnp.float32),
            pltpu.VMEM((640,), jnp.float32),
            pltpu.VMEM((NROW, HALF), jnp.float32),
            pltpu.VMEM_SHARED((N_AGG, HALF), jnp.float32),
            pltpu.VMEM_SHARED((N_AGG, DEGW), jnp.float32),
            pltpu.SemaphoreType.DMA,
            pltpu.SemaphoreType.DMA,
        ],
        compiler_params=pltpu.CompilerParams(use_tc_tiling_on_sc=False,
                                             needs_layout_passes=False),
    )
    return fn(h2, srcp, dstp)


def _linear_body(x_ref, w_ref, b_ref, o0_ref, o1_ref):
    h = jnp.dot(x_ref[...], w_ref[...], preferred_element_type=jnp.float32)
    h = jnp.maximum(h + b_ref[...], 0.0)
    o0_ref[...] = h[:, :HALF]
    o1_ref[...] = h[:, HALF:]


def _linear(x, W, b):
    B = 1000
    h0, h1 = pl.pallas_call(
        _linear_body,
        grid=(N // B,),
        in_specs=[pl.BlockSpec((B, D), lambda i: (i, 0)),
                  pl.BlockSpec((D, D), lambda i: (0, 0)),
                  pl.BlockSpec((1, D), lambda i: (0, 0))],
        out_specs=[pl.BlockSpec((B, HALF), lambda i: (i, 0)),
                   pl.BlockSpec((B, HALF), lambda i: (i, 0))],
        out_shape=[jax.ShapeDtypeStruct((N, HALF), jnp.float32)] * 2,
    )(x, W, b.reshape(1, D))
    return jnp.concatenate([h0, h1], axis=0)


def _lsm_body(z0_ref, z1_ref, o_ref):
    v = jnp.concatenate([z0_ref[...], z1_ref[...]], axis=1)
    m = jnp.max(v, axis=1, keepdims=True)
    lse = jnp.log(jnp.sum(jnp.exp(v - m), axis=1, keepdims=True)) + m
    o_ref[...] = v - lse


def _logsoftmax(z0, z1):
    B = 1000
    return pl.pallas_call(
        _lsm_body,
        grid=(N // B,),
        in_specs=[pl.BlockSpec((B, HALF), lambda i: (i, 0)),
                  pl.BlockSpec((B, HALF), lambda i: (i, 0))],
        out_specs=pl.BlockSpec((B, D), lambda i: (i, 0)),
        out_shape=jax.ShapeDtypeStruct((N, D), jnp.float32),
    )(z0, z1)


def kernel(x, edge_index, W, b):
    E = edge_index.shape[1]
    cpt = -(-E // (NS * CH))
    cpt += cpt % 2
    e_pad = NS * cpt * CH
    pad = e_pad - E
    pad_src = (jnp.arange(pad, dtype=jnp.int32) * 131) % N
    pad_dst = N + (jnp.arange(pad, dtype=jnp.int32) % 16)
    srcp = jnp.concatenate([edge_index[0], pad_src]).reshape(NS, cpt, CH)
    dstp = jnp.concatenate([edge_index[1], pad_dst]).reshape(NS, cpt, CH)

    h2 = _linear(x, W, b)
    z2, _ = _propagate(h2, srcp, dstp)
    return _logsoftmax(z2[:N], z2[N:])


# speedup vs baseline: 24.4957x; 1.1829x over previous
import functools

import jax
import jax.numpy as jnp
from jax import lax
from jax.experimental import pallas as pl
from jax.experimental.pallas import tpu as pltpu
from jax.experimental.pallas import tpu_sc as plsc

N = 10000
D = 128
HALF = 64
K = 50
ALPHA = 0.1
NS = 16
NC = 2
CH = 128
NPT = N // NS
NROW = 125
NCHUNK = NPT // NROW
N_AGG = 10240
DEGW = 16


def _rsqrt_newton(d):
    i = plsc.bitcast(d, jnp.int32)
    i = jnp.int32(0x5F3759DF) - (i >> 1)
    y = plsc.bitcast(i, jnp.float32)
    for _ in range(3):
        y = y * (1.5 - 0.5 * d * y * y)
    return y


def _prop_body(h_hbm, src_hbm, dst_hbm, z_hbm, zt_hbm,
               srcv, dstv, eb0, eb1, eb2, eb3, dinvv,
               hb, agg_sh, g0, g1, g2, g3, s0, s1, s2, s3):
    c = lax.axis_index("c")
    s = lax.axis_index("s")
    cN = c * N
    cpt = srcv.shape[0]
    ebs = (eb0, eb1, eb2, eb3)
    gsem = (g0, g1, g2, g3)
    ssem = (s0, s1, s2, s3)

    pltpu.sync_copy(src_hbm.at[s], srcv)
    pltpu.sync_copy(dst_hbm.at[s], dstv)

    @pl.loop(0, cpt)
    def _shift(j):
        for q in range(CH // 16):
            sl = pl.ds(q * 16, 16)
            srcv[j, sl] = srcv[j, sl] + cN

    zero = jnp.zeros((16,), jnp.float32)
    e0 = jnp.where(lax.iota(jnp.int32, 16) == 0, 1.0, 0.0)

    @pl.loop(0, CH)
    def _fill(i):
        for q in range(HALF // 16):
            eb0[i, pl.ds(q * 16, 16)] = zero
            eb1[i, pl.ds(q * 16, 16)] = e0 if q == 0 else zero

    @pl.loop(0, 5)
    def _zagg(k):
        pltpu.sync_copy(eb0, agg_sh.at[pl.ds(s * 640 + k * 128, 128)])

    plsc.subcore_barrier()

    @pl.loop(0, cpt)
    def _deg(j):
        pltpu.sync_copy(eb1, agg_sh.at[dstv.at[j]], add=True)

    plsc.subcore_barrier()

    lane = lax.iota(jnp.int32, 16)
    zlane = jnp.zeros((16,), jnp.int32)

    @pl.loop(0, 5)
    def _dinv(cc):
        pltpu.sync_copy(agg_sh.at[pl.ds(s * NPT + cc * 128, 128)], eb2)

        @pl.loop(0, 8)
        def _grp(g):
            dvals = plsc.load_gather(eb2, [g * 16 + lane, zlane]) + 1.0
            dinvv[pl.ds(cc * 128 + g * 16, 16)] = _rsqrt_newton(dvals)

    plsc.subcore_barrier()

    @pl.loop(0, 5)
    def _zagg2(k):
        pltpu.sync_copy(eb0, agg_sh.at[pl.ds(s * 640 + k * 128, 128)])

    @pl.loop(0, NCHUNK)
    def _init(k):
        r0 = s * NPT + k * NROW
        pltpu.sync_copy(h_hbm.at[pl.ds(cN + r0, NROW)], hb)

        @pl.loop(0, NROW)
        def _rows(i):
            dv = dinvv[pl.ds(k * NROW + i, 16)][0]
            for q in range(HALF // 16):
                sl = pl.ds(q * 16, 16)
                hb[i, sl] = dv * hb[i, sl]

        pltpu.sync_copy(hb, zt_hbm.at[pl.ds(cN + r0, NROW)])

    plsc.subcore_barrier()

    @pl.loop(0, K)
    def _step(t):
        for b in range(4):
            pltpu.async_copy(zt_hbm.at[srcv.at[b]], ebs[b], gsem[b])

        @pl.loop(0, cpt // 4)
        def _edges(kk):
            for b in range(4):
                j = 4 * kk + b
                pltpu.make_async_copy(zt_hbm.at[srcv.at[j]], ebs[b],
                                      gsem[b]).wait()
                pltpu.async_copy(ebs[b], agg_sh.at[dstv.at[j]], ssem[b],
                                 add=True)

                @pl.when(j + 4 < cpt)
                def _next():
                    pltpu.make_async_copy(ebs[b], agg_sh.at[dstv.at[j]],
                                          ssem[b]).wait()
                    pltpu.async_copy(zt_hbm.at[srcv.at[j + 4]], ebs[b],
                                     gsem[b])

        for b in range(4):
            pltpu.make_async_copy(ebs[b], agg_sh.at[dstv.at[cpt - 4 + b]],
                                  ssem[b]).wait()

        plsc.subcore_barrier()

        @pl.loop(0, NCHUNK)
        def _nodes(k):
            r0 = s * NPT + k * NROW
            pltpu.sync_copy(agg_sh.at[pl.ds(r0, NROW)], eb2.at[pl.ds(0, NROW)])
            pltpu.sync_copy(zt_hbm.at[pl.ds(cN + r0, NROW)], eb3.at[pl.ds(0, NROW)])
            pltpu.sync_copy(h_hbm.at[pl.ds(cN + r0, NROW)], hb)

            @pl.loop(0, NROW)
            def _rows(i):
                dv = dinvv[pl.ds(k * NROW + i, 16)][0]
                a9 = (1.0 - ALPHA) * dv
                for q in range(HALF // 16):
                    sl = pl.ds(q * 16, 16)
                    zv = a9 * (eb2[i, sl] + eb3[i, sl]) + ALPHA * hb[i, sl]
                    hb[i, sl] = zv
                    eb3[i, sl] = dv * zv
                    eb2[i, sl] = zero

            pltpu.sync_copy(eb3.at[pl.ds(0, NROW)], zt_hbm.at[pl.ds(cN + r0, NROW)])
            pltpu.sync_copy(hb, z_hbm.at[pl.ds(cN + r0, NROW)])
            pltpu.sync_copy(eb2.at[pl.ds(0, NROW)], agg_sh.at[pl.ds(r0, NROW)])

        plsc.subcore_barrier()


def _propagate(h2, srcp, dstp):
    cpt = srcp.shape[1]
    mesh = plsc.VectorSubcoreMesh(core_axis_name="c", subcore_axis_name="s")
    fn = pl.kernel(
        _prop_body,
        out_type=(jax.ShapeDtypeStruct((NC * N, HALF), jnp.float32),
                  jax.ShapeDtypeStruct((NC * N, HALF), jnp.float32)),
        mesh=mesh,
        scratch_types=[
            pltpu.VMEM((cpt, CH), jnp.int32),
            pltpu.VMEM((cpt, CH), jnp.int32),
            pltpu.VMEM((CH, HALF), jnp.float32),
            pltpu.VMEM((CH, HALF), jnp.float32),
            pltpu.VMEM((CH, HALF), jnp.float32),
            pltpu.VMEM((CH, HALF), jnp.float32),
            pltpu.VMEM((640,), jnp.float32),
            pltpu.VMEM((NROW, HALF), jnp.float32),
            pltpu.VMEM_SHARED((N_AGG, HALF), jnp.float32),
        ] + [pltpu.SemaphoreType.DMA] * 8,
        compiler_params=pltpu.CompilerParams(use_tc_tiling_on_sc=False,
                                             needs_layout_passes=False),
    )
    return fn(h2, srcp, dstp)


def _linear_body(x_ref, w_ref, b_ref, o0_ref, o1_ref):
    h = jnp.dot(x_ref[...], w_ref[...], preferred_element_type=jnp.float32)
    h = jnp.maximum(h + b_ref[...], 0.0)
    o0_ref[...] = h[:, :HALF]
    o1_ref[...] = h[:, HALF:]


def _linear(x, W, b):
    B = 1000
    h0, h1 = pl.pallas_call(
        _linear_body,
        grid=(N // B,),
        in_specs=[pl.BlockSpec((B, D), lambda i: (i, 0)),
                  pl.BlockSpec((D, D), lambda i: (0, 0)),
                  pl.BlockSpec((1, D), lambda i: (0, 0))],
        out_specs=[pl.BlockSpec((B, HALF), lambda i: (i, 0)),
                   pl.BlockSpec((B, HALF), lambda i: (i, 0))],
        out_shape=[jax.ShapeDtypeStruct((N, HALF), jnp.float32)] * 2,
    )(x, W, b.reshape(1, D))
    return jnp.concatenate([h0, h1], axis=0)


def _lsm_body(z0_ref, z1_ref, o_ref):
    v = jnp.concatenate([z0_ref[...], z1_ref[...]], axis=1)
    m = jnp.max(v, axis=1, keepdims=True)
    lse = jnp.log(jnp.sum(jnp.exp(v - m), axis=1, keepdims=True)) + m
    o_ref[...] = v - lse


def _logsoftmax(z0, z1):
    B = 1000
    return pl.pallas_call(
        _lsm_body,
        grid=(N // B,),
        in_specs=[pl.BlockSpec((B, HALF), lambda i: (i, 0)),
                  pl.BlockSpec((B, HALF), lambda i: (i, 0))],
        out_specs=pl.BlockSpec((B, D), lambda i: (i, 0)),
        out_shape=jax.ShapeDtypeStruct((N, D), jnp.float32),
    )(z0, z1)


def kernel(x, edge_index, W, b):
    E = edge_index.shape[1]
    cpt = -(-E // (NS * CH))
    cpt = -(-cpt // 4) * 4
    e_pad = NS * cpt * CH
    pad = e_pad - E
    pad_src = (jnp.arange(pad, dtype=jnp.int32) * 131) % N
    pad_dst = N + (jnp.arange(pad, dtype=jnp.int32) % 16)
    srcp = jnp.concatenate([edge_index[0], pad_src]).reshape(NS, cpt, CH)
    dstp = jnp.concatenate([edge_index[1], pad_dst]).reshape(NS, cpt, CH)

    h2 = _linear(x, W, b)
    z2, _ = _propagate(h2, srcp, dstp)
    return _logsoftmax(z2[:N], z2[N:])
